# baseline (device time: 278070 ns/iter reference)
import jax
import jax.numpy as jnp
from jax import lax
from jax.experimental import pallas as pl
from jax.experimental.pallas import tpu as pltpu

N_DEV = 32
HALF = N_DEV // 2


def kernel(x, Wq, K_ext, V_ext, Wo):
    B, Sq_l, E = x.shape
    H4 = Wq.shape[1] // 64
    R = B * Sq_l
    C = Wq.shape[1]

    xb = x.astype(jnp.bfloat16).reshape(R, E)
    wqt = Wq.T.astype(jnp.bfloat16)
    wob = Wo.astype(jnp.bfloat16)
    kt = K_ext.transpose(2, 0, 1, 3).astype(jnp.bfloat16)
    vt = V_ext.transpose(2, 0, 1, 3).astype(jnp.bfloat16)
    Skv = kt.shape[2]

    def body(x_ref, wqt_ref, k_ref, v_ref, wo_ref, out_ref,
             comm_f, comm_b, ctx_ref,
             send_f, recv_f, send_b, recv_b):
        my = lax.axis_index("i")
        left = lax.rem(my - 1 + N_DEV, N_DEV)
        right = lax.rem(my + 1, N_DEV)

        barrier_sem = pltpu.get_barrier_semaphore()
        for nbr in (left, right):
            pl.semaphore_signal(
                barrier_sem, inc=1,
                device_id=(nbr,), device_id_type=pl.DeviceIdType.MESH,
            )
        pl.semaphore_wait(barrier_sem, 2)

        def fwd_rdma(slot, other):
            return pltpu.make_async_remote_copy(
                src_ref=comm_f.at[slot], dst_ref=comm_f.at[other],
                send_sem=send_f.at[slot], recv_sem=recv_f.at[other],
                device_id=(right,), device_id_type=pl.DeviceIdType.MESH,
            )

        def bwd_rdma(src_ref, slot, other):
            return pltpu.make_async_remote_copy(
                src_ref=src_ref, dst_ref=comm_b.at[other],
                send_sem=send_b.at[slot], recv_sem=recv_b.at[other],
                device_id=(left,), device_id_type=pl.DeviceIdType.MESH,
            )

        def compute_chunk(wq_c, wo_c, hb):
            q = lax.dot_general(
                x_ref[...], wq_c, (((1,), (1,)), ((), ())),
                preferred_element_type=jnp.float32,
            )
            q = (q * 0.125).astype(jnp.bfloat16)
            kh = k_ref[pl.ds(hb * H4, H4)]
            vh = v_ref[pl.ds(hb * H4, H4)]
            for h in range(H4):
                for b in range(B):
                    qg = q[b * Sq_l:(b + 1) * Sq_l,
                           h * 64:(h + 1) * 64].reshape(4, 64, 64)
                    kg = kh[h, b].reshape(4, 64, 64)
                    vg = vh[h, b].reshape(4, 64, 64)
                    sc = lax.dot_general(
                        qg, kg, (((2,), (2,)), ((0,), (0,))),
                        preferred_element_type=jnp.float32,
                    )
                    m = jnp.max(sc, axis=-1, keepdims=True)
                    w = jnp.exp(sc - m)
                    w = (w / jnp.sum(w, axis=-1, keepdims=True)
                         ).astype(jnp.bfloat16)
                    ctx = lax.dot_general(
                        w, vg, (((2,), (1,)), ((0,), (0,))),
                        preferred_element_type=jnp.float32,
                    )
                    ctx_ref[b * Sq_l:(b + 1) * Sq_l, h * 64:(h + 1) * 64] = (
                        ctx.reshape(Sq_l, 64).astype(jnp.bfloat16))
            out_ref[...] += lax.dot_general(
                ctx_ref[...], wo_c, (((1,), (0,)), ((), ())),
                preferred_element_type=jnp.float32,
            )

        comm_f[0, 0] = wqt_ref[...]
        comm_f[0, 1] = wo_ref[...]
        out_ref[...] = jnp.zeros((R, E), jnp.float32)

        fwd_rdma(0, 1).start()
        bwd_rdma(comm_f.at[0], 0, 1).start()
        compute_chunk(comm_f[0, 0], comm_f[0, 1], my)
        fwd_rdma(0, 1).wait()
        bwd_rdma(comm_f.at[0], 0, 1).wait()

        def step(s, carry):
            slot = lax.rem(s, 2)
            other = 1 - slot

            fwd_rdma(slot, other).start()

            @pl.when(s < HALF - 1)
            def _():
                bwd_rdma(comm_b.at[slot], slot, other).start()

            compute_chunk(comm_f[slot, 0], comm_f[slot, 1],
                          lax.rem(my - s + N_DEV, N_DEV))
            compute_chunk(comm_b[slot, 0], comm_b[slot, 1],
                          lax.rem(my + s, N_DEV))

            fwd_rdma(slot, other).wait()

            @pl.when(s < HALF - 1)
            def _():
                bwd_rdma(comm_b.at[slot], slot, other).wait()

            return carry

        lax.fori_loop(1, HALF, step, 0)

        compute_chunk(comm_f[0, 0], comm_f[0, 1],
                      lax.rem(my + HALF, N_DEV))

    out = pl.pallas_call(
        body,
        out_shape=jax.ShapeDtypeStruct((R, E), jnp.float32),
        in_specs=[pl.BlockSpec(memory_space=pltpu.VMEM)] * 5,
        out_specs=pl.BlockSpec(memory_space=pltpu.VMEM),
        scratch_shapes=[
            pltpu.VMEM((2, 2, C, E), jnp.bfloat16),
            pltpu.VMEM((2, 2, C, E), jnp.bfloat16),
            pltpu.VMEM((R, C), jnp.bfloat16),
            pltpu.SemaphoreType.DMA((2,)),
            pltpu.SemaphoreType.DMA((2,)),
            pltpu.SemaphoreType.DMA((2,)),
            pltpu.SemaphoreType.DMA((2,)),
        ],
        compiler_params=pltpu.CompilerParams(collective_id=0),
    )(xb, wqt, kt, vt, wob)
    return out.reshape(B, Sq_l, E)


# device time: 250434 ns/iter; 1.1104x vs baseline; 1.1104x over previous
import jax
import jax.numpy as jnp
from jax import lax
from jax.experimental import pallas as pl
from jax.experimental.pallas import tpu as pltpu

N_DEV = 32
HALF = N_DEV // 2


def kernel(x, Wq, K_ext, V_ext, Wo):
    B, Sq_l, E = x.shape
    H4 = Wq.shape[1] // 64
    R = B * Sq_l
    C = Wq.shape[1]
    Skv = K_ext.shape[1]

    xb = x.astype(jnp.bfloat16).reshape(R, E)
    wqt = Wq.T.astype(jnp.bfloat16)
    wob = Wo.astype(jnp.bfloat16)
    H = K_ext.shape[2]
    kb16 = K_ext.reshape(B, Skv, H * 64).astype(jnp.bfloat16)
    vb16 = V_ext.reshape(B, Skv, H * 64).astype(jnp.bfloat16)

    def body(x_ref, wqt_ref, k_ref, v_ref, wo_ref, out_ref,
             comm_f, comm_b, kv_ref, ctx_ref,
             send_f, recv_f, send_b, recv_b, kv_sems):
        my = lax.axis_index("i")
        left = lax.rem(my - 1 + N_DEV, N_DEV)
        right = lax.rem(my + 1, N_DEV)

        barrier_sem = pltpu.get_barrier_semaphore()
        for nbr in (left, right):
            pl.semaphore_signal(
                barrier_sem, inc=1,
                device_id=(nbr,), device_id_type=pl.DeviceIdType.MESH,
            )
        pl.semaphore_wait(barrier_sem, 2)

        def fwd_rdma(slot, other):
            return pltpu.make_async_remote_copy(
                src_ref=comm_f.at[slot], dst_ref=comm_f.at[other],
                send_sem=send_f.at[slot], recv_sem=recv_f.at[other],
                device_id=(right,), device_id_type=pl.DeviceIdType.MESH,
            )

        def bwd_rdma(src_ref, slot, other):
            return pltpu.make_async_remote_copy(
                src_ref=src_ref, dst_ref=comm_b.at[other],
                send_sem=send_b.at[slot], recv_sem=recv_b.at[other],
                device_id=(left,), device_id_type=pl.DeviceIdType.MESH,
            )

        def kv_dmas(par, stream, hb):
            dmas = []
            for t, src in ((0, k_ref), (1, v_ref)):
                for b in range(B):
                    dmas.append(pltpu.make_async_copy(
                        src.at[b, :, pl.ds(hb * C, C)],
                        kv_ref.at[par, stream, t, b],
                        kv_sems.at[par, stream],
                    ))
            return dmas

        def kv_prefetch(par, stream, hb):
            for d in kv_dmas(par, stream, hb):
                d.start()

        def kv_wait(par, stream, hb):
            for d in kv_dmas(par, stream, hb):
                d.wait()

        def compute_chunk(slot, stream, par, hb):
            buf = comm_f if stream == 0 else comm_b
            wq_c = buf[slot, 0]
            wo_c = buf[slot, 1]
            q = lax.dot_general(
                x_ref[...], wq_c, (((1,), (1,)), ((), ())),
                preferred_element_type=jnp.float32,
            )
            q = (q * 0.125).astype(jnp.bfloat16)
            for b in range(B):
                k_all = kv_ref[par, stream, 0, b]
                v_all = kv_ref[par, stream, 1, b]
                for h in range(H4):
                    qg = q[b * Sq_l:(b + 1) * Sq_l,
                           h * 64:(h + 1) * 64].reshape(4, 64, 64)
                    kg = k_all[:, h * 64:(h + 1) * 64].reshape(4, 64, 64)
                    vg = v_all[:, h * 64:(h + 1) * 64].reshape(4, 64, 64)
                    sc = lax.dot_general(
                        qg, kg, (((2,), (2,)), ((0,), (0,))),
                        preferred_element_type=jnp.float32,
                    )
                    m = jnp.max(sc, axis=-1, keepdims=True)
                    w = jnp.exp(sc - m)
                    w = (w / jnp.sum(w, axis=-1, keepdims=True)
                         ).astype(jnp.bfloat16)
                    ctx = lax.dot_general(
                        w, vg, (((2,), (1,)), ((0,), (0,))),
                        preferred_element_type=jnp.float32,
                    )
                    ctx_ref[b * Sq_l:(b + 1) * Sq_l, h * 64:(h + 1) * 64] = (
                        ctx.reshape(Sq_l, 64).astype(jnp.bfloat16))
            out_ref[...] += lax.dot_general(
                ctx_ref[...], wo_c, (((1,), (0,)), ((), ())),
                preferred_element_type=jnp.float32,
            )

        def hb_f(s):
            return lax.rem(my - s + 2 * N_DEV, N_DEV)

        def hb_b(s):
            return lax.rem(my + s, N_DEV)

        comm_f[0, 0] = wqt_ref[...]
        comm_f[0, 1] = wo_ref[...]
        out_ref[...] = jnp.zeros((R, E), jnp.float32)

        kv_prefetch(0, 0, hb_f(0))
        kv_prefetch(1, 0, hb_f(1))
        kv_prefetch(1, 1, hb_b(1))

        fwd_rdma(0, 1).start()
        bwd_rdma(comm_f.at[0], 0, 1).start()
        kv_wait(0, 0, hb_f(0))
        compute_chunk(0, 0, 0, hb_f(0))
        fwd_rdma(0, 1).wait()
        bwd_rdma(comm_f.at[0], 0, 1).wait()

        def pair(i, carry):
            s1 = 2 * i + 1
            fwd_rdma(1, 0).start()
            bwd_rdma(comm_b.at[1], 1, 0).start()
            kv_prefetch(0, 0, hb_f(s1 + 1))
            kv_prefetch(0, 1, hb_b(s1 + 1))
            kv_wait(1, 0, hb_f(s1))
            compute_chunk(1, 0, 1, hb_f(s1))
            kv_wait(1, 1, hb_b(s1))
            compute_chunk(1, 1, 1, hb_b(s1))
            fwd_rdma(1, 0).wait()
            bwd_rdma(comm_b.at[1], 1, 0).wait()

            s2 = 2 * i + 2
            fwd_rdma(0, 1).start()
            bwd_rdma(comm_b.at[0], 0, 1).start()
            kv_prefetch(1, 0, hb_f(s2 + 1))
            kv_prefetch(1, 1, hb_b(s2 + 1))
            kv_wait(0, 0, hb_f(s2))
            compute_chunk(0, 0, 0, hb_f(s2))
            kv_wait(0, 1, hb_b(s2))
            compute_chunk(0, 1, 0, hb_b(s2))
            fwd_rdma(0, 1).wait()
            bwd_rdma(comm_b.at[0], 0, 1).wait()
            return carry

        lax.fori_loop(0, (HALF - 2) // 2, pair, 0)

        fwd_rdma(1, 0).start()
        kv_prefetch(0, 0, hb_f(16))
        kv_wait(1, 0, hb_f(15))
        compute_chunk(1, 0, 1, hb_f(15))
        kv_wait(1, 1, hb_b(15))
        compute_chunk(1, 1, 1, hb_b(15))
        fwd_rdma(1, 0).wait()

        kv_wait(0, 0, hb_f(16))
        compute_chunk(0, 0, 0, hb_f(16))

    out = pl.pallas_call(
        body,
        out_shape=jax.ShapeDtypeStruct((R, E), jnp.float32),
        in_specs=[pl.BlockSpec(memory_space=pltpu.VMEM)] * 5,
        out_specs=pl.BlockSpec(memory_space=pltpu.VMEM),
        scratch_shapes=[
            pltpu.VMEM((2, 2, C, E), jnp.bfloat16),
            pltpu.VMEM((2, 2, C, E), jnp.bfloat16),
            pltpu.VMEM((2, 2, 2, B, Skv, C), jnp.bfloat16),
            pltpu.VMEM((R, C), jnp.bfloat16),
            pltpu.SemaphoreType.DMA((2,)),
            pltpu.SemaphoreType.DMA((2,)),
            pltpu.SemaphoreType.DMA((2,)),
            pltpu.SemaphoreType.DMA((2,)),
            pltpu.SemaphoreType.DMA((2, 2)),
        ],
        compiler_params=pltpu.CompilerParams(collective_id=0),
    )(xb, wqt, kb16, vb16, wob)
    return out.reshape(B, Sq_l, E)


# device time: 247471 ns/iter; 1.1236x vs baseline; 1.0120x over previous
import jax
import jax.numpy as jnp
from jax import lax
from jax.experimental import pallas as pl
from jax.experimental.pallas import tpu as pltpu

N_DEV = 32
HALF = N_DEV // 2


def kernel(x, Wq, K_ext, V_ext, Wo):
    B, Sq_l, E = x.shape
    H4 = Wq.shape[1] // 64
    R = B * Sq_l
    C = Wq.shape[1]
    Skv = K_ext.shape[1]

    xb = x.astype(jnp.bfloat16).reshape(R, E)
    wqt = Wq.T.astype(jnp.bfloat16)
    wob = Wo.astype(jnp.bfloat16)
    H = K_ext.shape[2]
    kb16 = K_ext.reshape(B, Skv, H * 64).astype(jnp.bfloat16)
    vb16 = V_ext.reshape(B, Skv, H * 64).astype(jnp.bfloat16)

    def body(x_ref, wqt_ref, k_ref, v_ref, wo_ref, out_ref,
             comm_f, comm_b, kv_ref, ctx_ref,
             send_f, recv_f, send_b, recv_b, kv_sems):
        my = lax.axis_index("i")
        left = lax.rem(my - 1 + N_DEV, N_DEV)
        right = lax.rem(my + 1, N_DEV)

        barrier_sem = pltpu.get_barrier_semaphore()
        for nbr in (left, right):
            pl.semaphore_signal(
                barrier_sem, inc=1,
                device_id=(nbr,), device_id_type=pl.DeviceIdType.MESH,
            )
        pl.semaphore_wait(barrier_sem, 2)

        def fwd_rdma(slot, other):
            return pltpu.make_async_remote_copy(
                src_ref=comm_f.at[slot], dst_ref=comm_f.at[other],
                send_sem=send_f.at[slot], recv_sem=recv_f.at[other],
                device_id=(right,), device_id_type=pl.DeviceIdType.MESH,
            )

        def bwd_rdma(src_ref, slot, other):
            return pltpu.make_async_remote_copy(
                src_ref=src_ref, dst_ref=comm_b.at[other],
                send_sem=send_b.at[slot], recv_sem=recv_b.at[other],
                device_id=(left,), device_id_type=pl.DeviceIdType.MESH,
            )

        def kv_dmas(par, stream, hb):
            dmas = []
            for t, src in ((0, k_ref), (1, v_ref)):
                for b in range(B):
                    dmas.append(pltpu.make_async_copy(
                        src.at[b, :, pl.ds(hb * C, C)],
                        kv_ref.at[par, stream, t, b],
                        kv_sems.at[par, stream],
                    ))
            return dmas

        def kv_prefetch(par, stream, hb):
            for d in kv_dmas(par, stream, hb):
                d.start()

        def kv_wait(par, stream, hb):
            for d in kv_dmas(par, stream, hb):
                d.wait()

        def compute_chunk(slot, stream, par, hb):
            buf = comm_f if stream == 0 else comm_b
            wq_c = buf[slot, 0]
            wo_c = buf[slot, 1]
            q = lax.dot_general(
                x_ref[...], wq_c, (((1,), (1,)), ((), ())),
                preferred_element_type=jnp.float32,
            )
            q = (q * 0.125).astype(jnp.bfloat16)
            k_all = kv_ref[par, stream, 0]
            v_all = kv_ref[par, stream, 1]
            for h in range(H4):
                qh = q[:, h * 64:(h + 1) * 64].reshape(2 * 4, 64, 64)
                kh = k_all[:, :, h * 64:(h + 1) * 64].reshape(2 * 4, 64, 64)
                vh = v_all[:, :, h * 64:(h + 1) * 64].reshape(2 * 4, 64, 64)
                sc = lax.dot_general(
                    qh, kh, (((2,), (2,)), ((0,), (0,))),
                    preferred_element_type=jnp.float32,
                )
                m = jnp.max(sc, axis=-1, keepdims=True)
                w = jnp.exp(sc - m)
                w = (w / jnp.sum(w, axis=-1, keepdims=True)
                     ).astype(jnp.bfloat16)
                ctx = lax.dot_general(
                    w, vh, (((2,), (1,)), ((0,), (0,))),
                    preferred_element_type=jnp.float32,
                )
                ctx_ref[:, h * 64:(h + 1) * 64] = (
                    ctx.reshape(R, 64).astype(jnp.bfloat16))
            out_ref[...] += lax.dot_general(
                ctx_ref[...], wo_c, (((1,), (0,)), ((), ())),
                preferred_element_type=jnp.float32,
            )

        def hb_f(s):
            return lax.rem(my - s + 2 * N_DEV, N_DEV)

        def hb_b(s):
            return lax.rem(my + s, N_DEV)

        comm_f[0, 0] = wqt_ref[...]
        comm_f[0, 1] = wo_ref[...]
        out_ref[...] = jnp.zeros((R, E), jnp.float32)

        kv_prefetch(0, 0, hb_f(0))
        kv_prefetch(1, 0, hb_f(1))
        kv_prefetch(1, 1, hb_b(1))

        fwd_rdma(0, 1).start()
        bwd_rdma(comm_f.at[0], 0, 1).start()
        kv_wait(0, 0, hb_f(0))
        compute_chunk(0, 0, 0, hb_f(0))
        fwd_rdma(0, 1).wait()
        bwd_rdma(comm_f.at[0], 0, 1).wait()

        def pair(i, carry):
            s1 = 2 * i + 1
            fwd_rdma(1, 0).start()
            bwd_rdma(comm_b.at[1], 1, 0).start()
            kv_prefetch(0, 0, hb_f(s1 + 1))
            kv_prefetch(0, 1, hb_b(s1 + 1))
            kv_wait(1, 0, hb_f(s1))
            compute_chunk(1, 0, 1, hb_f(s1))
            kv_wait(1, 1, hb_b(s1))
            compute_chunk(1, 1, 1, hb_b(s1))
            fwd_rdma(1, 0).wait()
            bwd_rdma(comm_b.at[1], 1, 0).wait()

            s2 = 2 * i + 2
            fwd_rdma(0, 1).start()
            bwd_rdma(comm_b.at[0], 0, 1).start()
            kv_prefetch(1, 0, hb_f(s2 + 1))
            kv_prefetch(1, 1, hb_b(s2 + 1))
            kv_wait(0, 0, hb_f(s2))
            compute_chunk(0, 0, 0, hb_f(s2))
            kv_wait(0, 1, hb_b(s2))
            compute_chunk(0, 1, 0, hb_b(s2))
            fwd_rdma(0, 1).wait()
            bwd_rdma(comm_b.at[0], 0, 1).wait()
            return carry

        lax.fori_loop(0, (HALF - 2) // 2, pair, 0)

        fwd_rdma(1, 0).start()
        kv_prefetch(0, 0, hb_f(16))
        kv_wait(1, 0, hb_f(15))
        compute_chunk(1, 0, 1, hb_f(15))
        kv_wait(1, 1, hb_b(15))
        compute_chunk(1, 1, 1, hb_b(15))
        fwd_rdma(1, 0).wait()

        kv_wait(0, 0, hb_f(16))
        compute_chunk(0, 0, 0, hb_f(16))

    out = pl.pallas_call(
        body,
        out_shape=jax.ShapeDtypeStruct((R, E), jnp.float32),
        in_specs=[pl.BlockSpec(memory_space=pltpu.VMEM)] * 5,
        out_specs=pl.BlockSpec(memory_space=pltpu.VMEM),
        scratch_shapes=[
            pltpu.VMEM((2, 2, C, E), jnp.bfloat16),
            pltpu.VMEM((2, 2, C, E), jnp.bfloat16),
            pltpu.VMEM((2, 2, 2, B, Skv, C), jnp.bfloat16),
            pltpu.VMEM((R, C), jnp.bfloat16),
            pltpu.SemaphoreType.DMA((2,)),
            pltpu.SemaphoreType.DMA((2,)),
            pltpu.SemaphoreType.DMA((2,)),
            pltpu.SemaphoreType.DMA((2,)),
            pltpu.SemaphoreType.DMA((2, 2)),
        ],
        compiler_params=pltpu.CompilerParams(collective_id=0),
    )(xb, wqt, kb16, vb16, wob)
    return out.reshape(B, Sq_l, E)
